# Initial kernel scaffold; baseline (speedup 1.0000x reference)
import jax
import jax.numpy as jnp
from jax.experimental import pallas as pl


def kernel(x, edge_index, W1, b1, W2, b2):
    raise NotImplementedError("write your pallas kernel here")


# speedup vs baseline: 26.4764x; 26.4764x over previous
import functools

import jax
import jax.numpy as jnp
from jax import lax
from jax.experimental import pallas as pl
from jax.experimental.pallas import tpu as pltpu
from jax.experimental.pallas import tpu_sc as plsc

N = 10000
E = 320000
NP = 10240
NW = 32
CHUNK = 128
CPT = 79
EPAD = NW * CPT * CHUNK
RPT = NP // 16
F1 = 16
F2 = 48
BLK = 1000

_MESH = plsc.VectorSubcoreMesh(core_axis_name="c", subcore_axis_name="s")
_SC_PARAMS = pltpu.CompilerParams(use_tc_tiling_on_sc=False)


def _zero_rows(vbuf, nrows, ncols):
    zero = jnp.zeros((16,), jnp.float32)

    def body(i, carry):
        for k in range(ncols // 16):
            vbuf[i, pl.ds(16 * k, 16)] = zero
        return carry

    lax.fori_loop(0, nrows, body, 0)


def _hist_body(dst_hbm, out_hbm, dst_v, ones_v, vbuf, acc_sh):
    c = lax.axis_index("c")
    s = lax.axis_index("s")
    w = s * 2 + c
    pltpu.sync_copy(dst_hbm.at[w], dst_v)
    onehot = jnp.where(lax.iota(jnp.int32, 16) == 0, 1.0, 0.0)

    def obody(i, carry):
        ones_v[i, :] = onehot
        return carry

    lax.fori_loop(0, CHUNK, obody, 0)
    _zero_rows(vbuf, RPT, F1)
    pltpu.sync_copy(vbuf, acc_sh.at[pl.ds(s * RPT, RPT)])
    plsc.subcore_barrier()

    def ebody(j, carry):
        pltpu.sync_copy(ones_v, acc_sh.at[dst_v.at[j]], add=True)
        return carry

    lax.fori_loop(0, CPT, ebody, 0)
    plsc.subcore_barrier()
    pltpu.sync_copy(acc_sh.at[pl.ds(s * RPT, RPT)], vbuf)
    pltpu.sync_copy(vbuf, out_hbm.at[c, pl.ds(s * RPT, RPT)])


_sc_hist = functools.partial(
    pl.kernel,
    mesh=_MESH,
    compiler_params=_SC_PARAMS,
    out_type=jax.ShapeDtypeStruct((2, NP, F1), jnp.float32),
    scratch_types=[
        pltpu.VMEM((CPT, CHUNK), jnp.int32),
        pltpu.VMEM((CHUNK, F1), jnp.float32),
        pltpu.VMEM((RPT, F1), jnp.float32),
        pltpu.VMEM_SHARED((NP, F1), jnp.float32),
    ],
)(_hist_body)


def _scat_body(F, src_hbm, dst_hbm, tab_hbm, out_hbm, src_v, dst_v, rows_v,
               vbuf, acc_sh, sem):
    c = lax.axis_index("c")
    s = lax.axis_index("s")
    w = s * 2 + c
    pltpu.sync_copy(src_hbm.at[w], src_v)
    pltpu.sync_copy(dst_hbm.at[w], dst_v)
    _zero_rows(vbuf, RPT, F)
    pltpu.sync_copy(vbuf, acc_sh.at[pl.ds(s * RPT, RPT)])
    plsc.subcore_barrier()

    def ebody(j, carry):
        pltpu.async_copy(tab_hbm.at[src_v.at[j]], rows_v, sem).wait()
        pltpu.sync_copy(rows_v, acc_sh.at[dst_v.at[j]], add=True)
        return carry

    lax.fori_loop(0, CPT, ebody, 0)
    plsc.subcore_barrier()
    pltpu.sync_copy(acc_sh.at[pl.ds(s * RPT, RPT)], vbuf)
    pltpu.sync_copy(vbuf, out_hbm.at[c, pl.ds(s * RPT, RPT)])


def _make_scat(F):
    return functools.partial(
        pl.kernel,
        mesh=_MESH,
        compiler_params=_SC_PARAMS,
        out_type=jax.ShapeDtypeStruct((2, NP, F), jnp.float32),
        scratch_types=[
            pltpu.VMEM((CPT, CHUNK), jnp.int32),
            pltpu.VMEM((CPT, CHUNK), jnp.int32),
            pltpu.VMEM((CHUNK, F), jnp.float32),
            pltpu.VMEM((RPT, F), jnp.float32),
            pltpu.VMEM_SHARED((NP, F), jnp.float32),
            pltpu.SemaphoreType.DMA,
        ],
    )(functools.partial(_scat_body, F))


_sc_scat16 = _make_scat(F1)
_sc_scat48 = _make_scat(F2)


def _tc1_body(degp_ref, x_ref, w1_ref, hs_ref, dinv_ref):
    deg = degp_ref[0, :, 0:1] + degp_ref[1, :, 0:1] + 1.0
    dinv = lax.rsqrt(jnp.maximum(deg, 1e-12))
    h = jnp.dot(x_ref[...], w1_ref[...], preferred_element_type=jnp.float32)
    hs_ref[...] = h * dinv
    dinv_ref[...] = dinv


def _tc1(degp, x, W1):
    return pl.pallas_call(
        _tc1_body,
        grid=(N // BLK,),
        in_specs=[
            pl.BlockSpec((2, BLK, F1), lambda m: (0, m, 0)),
            pl.BlockSpec((BLK, 128), lambda m: (m, 0)),
            pl.BlockSpec((128, F1), lambda m: (0, 0)),
        ],
        out_specs=[
            pl.BlockSpec((BLK, F1), lambda m: (m, 0)),
            pl.BlockSpec((BLK, 1), lambda m: (m, 0)),
        ],
        out_shape=[
            jax.ShapeDtypeStruct((N, F1), jnp.float32),
            jax.ShapeDtypeStruct((N, 1), jnp.float32),
        ],
    )(degp, x, W1)


def _tc2_body(s1p_ref, hs1_ref, dinv_ref, b1_ref, w2p_ref, hs2_ref):
    dinv = dinv_ref[...]
    agg = dinv * (s1p_ref[0] + s1p_ref[1] + hs1_ref[...]) + b1_ref[...]
    g = jnp.maximum(agg, 0.0)
    h2 = jnp.dot(g, w2p_ref[...], preferred_element_type=jnp.float32)
    hs2_ref[...] = h2 * dinv


def _tc2(s1p, hs1, dinv, b1, W2p):
    return pl.pallas_call(
        _tc2_body,
        grid=(N // BLK,),
        in_specs=[
            pl.BlockSpec((2, BLK, F1), lambda m: (0, m, 0)),
            pl.BlockSpec((BLK, F1), lambda m: (m, 0)),
            pl.BlockSpec((BLK, 1), lambda m: (m, 0)),
            pl.BlockSpec((1, F1), lambda m: (0, 0)),
            pl.BlockSpec((F1, F2), lambda m: (0, 0)),
        ],
        out_specs=pl.BlockSpec((BLK, F2), lambda m: (m, 0)),
        out_shape=jax.ShapeDtypeStruct((N, F2), jnp.float32),
    )(s1p, hs1, dinv, b1, W2p)


def _tc3_body(s2p_ref, hs2_ref, dinv_ref, b2p_ref, out_ref):
    a = dinv_ref[...] * (s2p_ref[0] + s2p_ref[1] + hs2_ref[...]) + b2p_ref[...]
    a = a[:, :40]
    m = jnp.max(a, axis=1, keepdims=True)
    z = a - m
    lse = jnp.log(jnp.sum(jnp.exp(z), axis=1, keepdims=True))
    out_ref[...] = z - lse


def _tc3(s2p, hs2, dinv, b2p):
    return pl.pallas_call(
        _tc3_body,
        grid=(N // BLK,),
        in_specs=[
            pl.BlockSpec((2, BLK, F2), lambda m: (0, m, 0)),
            pl.BlockSpec((BLK, F2), lambda m: (m, 0)),
            pl.BlockSpec((BLK, 1), lambda m: (m, 0)),
            pl.BlockSpec((1, F2), lambda m: (0, 0)),
        ],
        out_specs=pl.BlockSpec((BLK, 40), lambda m: (m, 0)),
        out_shape=jax.ShapeDtypeStruct((N, 40), jnp.float32),
    )(s2p, hs2, dinv, b2p)


def kernel(x, edge_index, W1, b1, W2, b2):
    ei = edge_index.astype(jnp.int32)
    pad = EPAD - E
    src_p = jnp.concatenate([ei[0], jnp.zeros((pad,), jnp.int32)])
    dst_p = jnp.concatenate([ei[1], jnp.full((pad,), NP - 1, jnp.int32)])
    src_p = src_p.reshape(NW, CPT, CHUNK)
    dst_p = dst_p.reshape(NW, CPT, CHUNK)
    W2p = jnp.pad(W2, ((0, 0), (0, F2 - 40)))
    b2p = jnp.pad(b2, (0, F2 - 40)).reshape(1, F2)

    degp = _sc_hist(dst_p)
    hs1, dinv = _tc1(degp, x, W1)
    s1p = _sc_scat16(src_p, dst_p, hs1)
    hs2 = _tc2(s1p, hs1, dinv, b1.reshape(1, F1), W2p)
    s2p = _sc_scat48(src_p, dst_p, hs2)
    return _tc3(s2p, hs2, dinv, b2p)
